# baseline (speedup 1.0000x reference)
import functools

import jax
import jax.numpy as jnp
from jax import lax
from jax.experimental import pallas as pl
from jax.experimental.pallas import tpu as pltpu
from jax.experimental.pallas import tpu_sc as plsc

DIM = 32
B = 1024
NUM_GROUPS = 32
L = 16


def kernel(group_inputs, item_inputs, groups_members, user_table, item_table,
           W_att1, b_att1, W_att2, b_att2, W_p1, b_p1, W_p2, b_p2):
    info = plsc.get_sparse_core_info()
    NW = info.num_cores * info.num_subcores
    SPW = B // NW

    gi = group_inputs.astype(jnp.int32)
    ii = item_inputs.astype(jnp.int32)
    gm = groups_members.astype(jnp.int32).reshape(-1)
    W2T = W_att2.T
    Wp1p = jnp.pad(W_p1, ((0, 0), (0, L - 8)))
    wp2 = jnp.pad(W_p2[:, 0], (0, L - 8))
    b2p = jnp.pad(b_att2, (0, L - 3))
    bp1p = jnp.pad(b_p1, (0, L - 8))
    bp2p = jnp.pad(b_p2, (0, L - 1))

    mesh = plsc.VectorSubcoreMesh(core_axis_name="c", subcore_axis_name="s")

    @functools.partial(
        pl.kernel,
        out_type=jax.ShapeDtypeStruct((B,), jnp.float32),
        mesh=mesh,
        compiler_params=pltpu.CompilerParams(
            needs_layout_passes=False, use_tc_tiling_on_sc=False),
        scratch_types=[
            pltpu.VMEM((SPW,), jnp.int32),
            pltpu.VMEM((SPW,), jnp.int32),
            pltpu.VMEM((NUM_GROUPS * 3,), jnp.int32),
            pltpu.VMEM((3 * SPW,), jnp.int32),
            pltpu.VMEM((3 * SPW, DIM), jnp.float32),
            pltpu.VMEM((SPW, DIM), jnp.float32),
            pltpu.VMEM((4 * DIM, L), jnp.float32),
            pltpu.VMEM((3, L), jnp.float32),
            pltpu.VMEM((3 * DIM, L), jnp.float32),
            pltpu.VMEM((L,), jnp.float32),
            pltpu.VMEM((L,), jnp.float32),
            pltpu.VMEM((L,), jnp.float32),
            pltpu.VMEM((L,), jnp.float32),
            pltpu.VMEM((L,), jnp.float32),
            pltpu.VMEM((SPW,), jnp.float32),
            pltpu.SemaphoreType.DMA,
            pltpu.SemaphoreType.DMA,
        ],
    )
    def sc_kernel(g_hbm, i_hbm, gm_hbm, user_hbm, item_hbm, W1_hbm, W2T_hbm,
                  Wp1_hbm, b1_hbm, b2_hbm, bp1_hbm, wp2_hbm, bp2_hbm, out_hbm,
                  g_v, i_v, gm_v, mid_v, mrows, irows, W1_v, W2T_v, Wp1_v,
                  b1_v, b2_v, bp1_v, wp2_v, bp2_v, out_v, sem0, sem1):
        wid = lax.axis_index("s") * info.num_cores + lax.axis_index("c")
        base = wid * SPW

        pltpu.sync_copy(g_hbm.at[pl.ds(base, SPW)], g_v)
        pltpu.sync_copy(i_hbm.at[pl.ds(base, SPW)], i_v)
        pltpu.sync_copy(gm_hbm, gm_v)
        pltpu.sync_copy(W1_hbm, W1_v)
        pltpu.sync_copy(W2T_hbm, W2T_v)
        pltpu.sync_copy(Wp1_hbm, Wp1_v)
        pltpu.sync_copy(b1_hbm, b1_v)
        pltpu.sync_copy(b2_hbm, b2_v)
        pltpu.sync_copy(bp1_hbm, bp1_v)
        pltpu.sync_copy(wp2_hbm, wp2_v)
        pltpu.sync_copy(bp2_hbm, bp2_v)

        for grp in range(SPW // L):
            gl = g_v[pl.ds(grp * L, L)]
            for k in range(3):
                mk = plsc.load_gather(gm_v, [gl * 3 + k])
                mid_v[pl.ds(k * SPW + grp * L, L)] = mk

        cm = pltpu.async_copy(user_hbm.at[mid_v], mrows, sem0)
        ci = pltpu.async_copy(item_hbm.at[i_v], irows, sem1)
        cm.wait()
        ci.wait()

        zero = jnp.zeros((L,), jnp.float32)
        b1vec = b1_v[...]
        b2vec = b2_v[...]
        bp1vec = bp1_v[...]
        wp2vec = wp2_v[...]
        bp2vec = bp2_v[...]
        w2t = [W2T_v[0, :], W2T_v[1, :], W2T_v[2, :]]
        lane0 = lax.broadcasted_iota(jnp.int32, (L,), 0) == 0

        def body(b, carry):
            ih = [irows[b, pl.ds(0, L)], irows[b, pl.ds(L, L)]]
            mh = [[mrows[k * SPW + b, pl.ds(half * L, L)] for half in range(2)]
                  for k in range(3)]

            acc = [b1vec, zero, zero, zero]
            for k in range(3):
                for half in range(2):
                    for d in range(L):
                        i = 32 * k + L * half + d
                        acc[i % 4] = acc[i % 4] + mh[k][half][d] * W1_v[i, :]
            for half in range(2):
                for d in range(L):
                    i = 96 + L * half + d
                    acc[i % 4] = acc[i % 4] + ih[half][d] * W1_v[i, :]
            h = jnp.maximum((acc[0] + acc[1]) + (acc[2] + acc[3]), 0.0)

            lv = [(zero + jnp.sum(h * w2t[k])) + b2vec[k] for k in range(3)]
            mx = jnp.maximum(jnp.maximum(lv[0], lv[1]), lv[2])
            e = [jnp.exp(v - mx) for v in lv]
            ssum = (e[0] + e[1]) + e[2]
            w = [v / ssum for v in e]

            ge = [(w[0] * mh[0][half] + w[1] * mh[1][half]) + w[2] * mh[2][half]
                  for half in range(2)]
            el = [ge[half] * ih[half] for half in range(2)]

            acc2 = [bp1vec, zero, zero, zero]
            srcs = [el[0], el[1], ge[0], ge[1], ih[0], ih[1]]
            for j, vec in enumerate(srcs):
                for d in range(L):
                    i = L * j + d
                    acc2[i % 4] = acc2[i % 4] + vec[d] * Wp1_v[i, :]
            h2 = jnp.maximum((acc2[0] + acc2[1]) + (acc2[2] + acc2[3]), 0.0)

            yv = (zero + jnp.sum(h2 * wp2vec)) + bp2vec[0]
            sig = 1.0 / (1.0 + jnp.exp(-yv))
            plsc.store_scatter(out_v, [jnp.full((L,), b, jnp.int32)], sig,
                               mask=lane0)
            return carry

        lax.fori_loop(0, SPW, body, 0)
        pltpu.sync_copy(out_v, out_hbm.at[pl.ds(base, SPW)])

    y = sc_kernel(gi, ii, gm, user_table, item_table, W_att1, W2T, Wp1p,
                  b_att1, b2p, bp1p, wp2, bp2p)
    return y.reshape(B, 1)
